# baseline (speedup 1.0000x reference)
import jax
import jax.numpy as jnp
from jax import lax
from jax.experimental import pallas as pl
from jax.experimental.pallas import tpu as pltpu
from jax.experimental.pallas import tpu_sc as plsc

D_MODEL = 64
MAX_LEN = 8192
N = 819200
NC, NS = 2, 16
NW = NC * NS
B_PER_W = N // NW
G = 128
K = B_PER_W // G


def _sc_gather(x2, pe3):
    mesh = plsc.VectorSubcoreMesh(core_axis_name="c", subcore_axis_name="s")

    def body(x_hbm, pe_hbm, out_hbm, idx_v, rows_v, spm, sem):
        sid = lax.axis_index("s")
        wid = sid * NC + lax.axis_index("c")

        @pl.when(sid == 0)
        def _():
            pltpu.sync_copy(pe_hbm, spm)

        plsc.subcore_barrier()

        pltpu.sync_copy(x_hbm.at[pl.ds(wid * K, K)], idx_v)
        out_base = wid * K * G

        def step(t, _):
            pltpu.async_copy(pe_hbm.at[idx_v.at[t]], rows_v, sem).wait()
            pltpu.sync_copy(rows_v, out_hbm.at[pl.ds(out_base + t * G, G)])
            return ()

        lax.fori_loop(0, K, step, (), unroll=False)

    f = pl.kernel(
        body,
        out_type=jax.ShapeDtypeStruct((N, 1, D_MODEL), jnp.float32),
        mesh=mesh,
        scratch_types=[
            pltpu.VMEM((K, G), jnp.int32),
            pltpu.VMEM((G, 1, D_MODEL), jnp.float32),
            pltpu.VMEM_SHARED((MAX_LEN, 1, D_MODEL), jnp.float32),
            pltpu.SemaphoreType.DMA,
        ],
    )
    return f(x2, pe3)


def kernel(x, pe):
    x2 = x.astype(jnp.int32).reshape(NW * K, G)
    return _sc_gather(x2, pe[:, None, :])
